# baseline (speedup 1.0000x reference)
import functools

import jax
import jax.numpy as jnp
from jax import lax
from jax.experimental import pallas as pl
from jax.experimental.pallas import tpu as pltpu
from jax.experimental.pallas import tpu_sc as plsc

UST_DIM = 8
_NS, _LANES = 16, 16


def _make_sc_scale(F):
    num_cores = 1
    q_per_w = F // (num_cores * _NS)
    n_qv = q_per_w // _LANES
    mesh = plsc.VectorSubcoreMesh(
        core_axis_name="c", subcore_axis_name="s", num_cores=num_cores
    )

    @functools.partial(
        pl.kernel,
        mesh=mesh,
        out_type=jax.ShapeDtypeStruct((F,), jnp.float32),
        scratch_types=[pltpu.VMEM((q_per_w,), jnp.float32)],
    )
    def scale_sc(out_hbm, buf_v):
        wid = lax.axis_index("s") * num_cores + lax.axis_index("c")
        base = wid * q_per_w
        lane = lax.iota(jnp.int32, _LANES)
        for qv in range(n_qv):
            qf = (base + qv * _LANES + lane).astype(jnp.float32)

            def nbody(step, carry, qf=qf):
                mind, mini = carry
                for u in range(2):
                    n = step * 2 + u
                    diff = qf - n.astype(jnp.float32)
                    dist = jnp.float32(UST_DIM) * (diff * diff)
                    better = dist < mind
                    mind = jnp.where(better, dist, mind)
                    mini = jnp.where(better, n, mini)
                return mind, mini

            mind0 = jnp.full((_LANES,), jnp.float32(3.4e38))
            mini0 = jnp.zeros((_LANES,), jnp.int32)
            _, mini = lax.fori_loop(0, F // 2, nbody, (mind0, mini0))
            buf_v[pl.ds(qv * _LANES, _LANES)] = (
                mini.astype(jnp.float32) + 1.0
            ) / jnp.float32(F)
        pltpu.sync_copy(buf_v, out_hbm.at[pl.ds(base, q_per_w)])

    return scale_sc


def _fused_kernel(x_ref, o_ref, scale_ref):
    F = x_ref.shape[1]

    @pl.when(pl.program_id(0) == 0)
    def _compute_scale():
        qi = jax.lax.broadcasted_iota(jnp.int32, (F, F), 0)
        pj = jax.lax.broadcasted_iota(jnp.int32, (F, F), 1)
        diff = (qi - pj).astype(jnp.float32)
        dists = jnp.float32(UST_DIM) * (diff * diff)
        idx = jnp.argmin(dists, axis=1)
        scale_ref[...] = ((idx.astype(jnp.float32) + 1.0) / jnp.float32(F))[None, :]

    o_ref[...] = x_ref[...] * scale_ref[...]


def _mul2_kernel(dst_ref, x_ref, scale_ref, o_ref):
    del dst_ref
    o_ref[...] = x_ref[...] * scale_ref[...]


def kernel(inputs):
    B, F = inputs.shape
    BLK = 2048
    B1 = 8192

    scale_sc = _make_sc_scale(F)()

    out_a = pl.pallas_call(
        _fused_kernel,
        grid=(B1 // BLK,),
        in_specs=[pl.BlockSpec((BLK, F), lambda i: (i, 0))],
        out_specs=pl.BlockSpec((BLK, F), lambda i: (i, 0)),
        out_shape=jax.ShapeDtypeStruct((B, F), inputs.dtype),
        scratch_shapes=[pltpu.VMEM((1, F), jnp.float32)],
        compiler_params=pltpu.CompilerParams(
            dimension_semantics=("arbitrary",),
        ),
    )(inputs)

    n2 = (B - B1) // BLK
    off = B1 // BLK
    out = pl.pallas_call(
        _mul2_kernel,
        grid=(n2,),
        in_specs=[
            pl.BlockSpec(memory_space=pl.ANY),
            pl.BlockSpec((BLK, F), lambda i, off=off: (off + i, 0)),
            pl.BlockSpec((1, F), lambda i: (0, 0)),
        ],
        out_specs=pl.BlockSpec((BLK, F), lambda i, off=off: (off + i, 0)),
        out_shape=jax.ShapeDtypeStruct((B, F), inputs.dtype),
        input_output_aliases={0: 0},
        compiler_params=pltpu.CompilerParams(
            dimension_semantics=("parallel",),
        ),
    )(out_a, inputs, scale_sc.reshape(1, F))
    return out


# speedup vs baseline: 1.0031x; 1.0031x over previous
import functools

import jax
import jax.numpy as jnp
from jax import lax
from jax.experimental import pallas as pl
from jax.experimental.pallas import tpu as pltpu
from jax.experimental.pallas import tpu_sc as plsc

UST_DIM = 8
_NS, _LANES = 16, 16


def _make_sc_scale(F):
    num_cores = 1
    q_per_w = F // (num_cores * _NS)
    n_qv = q_per_w // _LANES
    mesh = plsc.VectorSubcoreMesh(
        core_axis_name="c", subcore_axis_name="s", num_cores=num_cores
    )

    @functools.partial(
        pl.kernel,
        mesh=mesh,
        out_type=jax.ShapeDtypeStruct((F,), jnp.float32),
        scratch_types=[pltpu.VMEM((q_per_w,), jnp.float32)],
    )
    def scale_sc(out_hbm, buf_v):
        wid = lax.axis_index("s") * num_cores + lax.axis_index("c")
        base = wid * q_per_w
        lane = lax.iota(jnp.int32, _LANES)
        for qv in range(n_qv):
            qf = (base + qv * _LANES + lane).astype(jnp.float32)

            def nbody(step, carry, qf=qf):
                mind, mini = carry
                for u in range(2):
                    n = step * 2 + u
                    diff = qf - n.astype(jnp.float32)
                    dist = jnp.float32(UST_DIM) * (diff * diff)
                    better = dist < mind
                    mind = jnp.where(better, dist, mind)
                    mini = jnp.where(better, n, mini)
                return mind, mini

            mind0 = jnp.full((_LANES,), jnp.float32(3.4e38))
            mini0 = jnp.zeros((_LANES,), jnp.int32)
            _, mini = lax.fori_loop(0, F // 2, nbody, (mind0, mini0))
            buf_v[pl.ds(qv * _LANES, _LANES)] = (
                mini.astype(jnp.float32) + 1.0
            ) / jnp.float32(F)
        pltpu.sync_copy(buf_v, out_hbm.at[pl.ds(base, q_per_w)])

    return scale_sc


def _fused_kernel(x_ref, o_ref, scale_ref):
    F = x_ref.shape[1]

    @pl.when(pl.program_id(0) == 0)
    def _compute_scale():
        qi = jax.lax.broadcasted_iota(jnp.int32, (F, F), 0)
        pj = jax.lax.broadcasted_iota(jnp.int32, (F, F), 1)
        diff = (qi - pj).astype(jnp.float32)
        dists = jnp.float32(UST_DIM) * (diff * diff)
        idx = jnp.argmin(dists, axis=1)
        scale_ref[...] = ((idx.astype(jnp.float32) + 1.0) / jnp.float32(F))[None, :]

    o_ref[...] = x_ref[...] * scale_ref[...]


def _mul2_kernel(dst_ref, x_ref, scale_ref, o_ref):
    del dst_ref
    o_ref[...] = x_ref[...] * scale_ref[...]


def kernel(inputs):
    B, F = inputs.shape
    BLK = 2048
    B1 = 8192

    scale_sc = _make_sc_scale(F)()

    out_a = pl.pallas_call(
        _fused_kernel,
        grid=(B1 // BLK,),
        in_specs=[pl.BlockSpec((BLK, F), lambda i: (i, 0))],
        out_specs=pl.BlockSpec((BLK, F), lambda i: (i, 0)),
        out_shape=jax.ShapeDtypeStruct((B, F), inputs.dtype),
        scratch_shapes=[pltpu.VMEM((1, F), jnp.float32)],
        compiler_params=pltpu.CompilerParams(
            dimension_semantics=("arbitrary",),
        ),
    )(inputs)

    n2 = (B - B1) // BLK
    off = B1 // BLK
    out = pl.pallas_call(
        _mul2_kernel,
        grid=(n2,),
        in_specs=[
            pl.BlockSpec(memory_space=pl.ANY),
            pl.BlockSpec((BLK, F), lambda i, off=off: (off + i, 0)),
            pl.BlockSpec((1, F), lambda i: (0, 0)),
        ],
        out_specs=pl.BlockSpec((BLK, F), lambda i, off=off: (off + i, 0)),
        out_shape=jax.ShapeDtypeStruct((B, F), inputs.dtype),
        input_output_aliases={0: 0},
        compiler_params=pltpu.CompilerParams(
            dimension_semantics=("arbitrary",),
        ),
    )(out_a, inputs, scale_sc.reshape(1, F))
    return out
